# baseline (speedup 1.0000x reference)
import functools

import jax
import jax.numpy as jnp
from jax import lax
from jax.experimental import pallas as pl
from jax.experimental.pallas import tpu as pltpu
from jax.experimental.pallas import tpu_sc as plsc

N = 10000
E = 320000
D = 128
NC = 40
NH = 128
NL = 4

ROWS = 400
GRID = N // ROWS

NSC = 2
NSUB = 16
NW = NSC * NSUB
EPW = E // NW
B = 128
EPP = 10240
NBT = EPP // B
NQ = NBT // 4
NA = 10240
RPS = NA // NSUB


def _spmm_body(W, xw, srcs, dsts, zeros, out, is0, id0, is1, id1, rows0,
               rows1, acc, gsem, ss0, ss1):
    c = lax.axis_index("c")
    s = lax.axis_index("s")
    wid = c * NSUB + s
    row0 = s * RPS
    base = wid * EPP
    idxs = [is0, is1]
    idxd = [id0, id1]
    rows = [rows0, rows1]
    ssem = [ss0, ss1]
    pltpu.sync_copy(zeros.at[pl.ds(row0, RPS)], acc.at[pl.ds(row0, RPS)])
    plsc.subcore_barrier()

    def step(j, carry):
        off = base + j * B
        pltpu.sync_copy(srcs.at[pl.ds(off, B)], idxs[0])
        pltpu.sync_copy(dsts.at[pl.ds(off, B)], idxd[0])
        pltpu.async_copy(xw.at[idxs[0]], rows[0], gsem).wait()
        pltpu.sync_copy(rows[0], acc.at[idxd[0]], add=True)
        return carry

    lax.fori_loop(0, NBT, step, 0)
    plsc.subcore_barrier()
    pltpu.sync_copy(acc.at[pl.ds(row0, RPS)], out.at[c, pl.ds(row0, RPS)])


@functools.partial(jax.jit, static_argnames=("W",))
def _spmm(xw, srcs, dsts, zeros, W):
    mesh = plsc.VectorSubcoreMesh(core_axis_name="c", subcore_axis_name="s")
    body = functools.partial(_spmm_body, W)
    return pl.kernel(
        body,
        out_type=jax.ShapeDtypeStruct((NSC, NA, W), jnp.float32),
        mesh=mesh,
        scratch_types=[
            pltpu.VMEM((B,), jnp.int32),
            pltpu.VMEM((B,), jnp.int32),
            pltpu.VMEM((B,), jnp.int32),
            pltpu.VMEM((B,), jnp.int32),
            pltpu.VMEM((B, W), jnp.float32),
            pltpu.VMEM((B, W), jnp.float32),
            pltpu.VMEM_SHARED((NA, W), jnp.float32),
            pltpu.SemaphoreType.DMA,
            pltpu.SemaphoreType.DMA,
            pltpu.SemaphoreType.DMA,
        ],
        name=f"sc_spmm_{W}",
    )(xw, srcs, dsts, zeros)


def _xc_body(x_ref, idxl_ref, labsel_ref, o_ref):
    i = pl.program_id(0)
    rid = lax.broadcasted_iota(jnp.int32, (ROWS, 1000), 0) + i * ROWS
    m1 = (rid == idxl_ref[...][None, :]).astype(jnp.float32)
    cid = lax.broadcasted_iota(jnp.int32, (1000, NC), 1)
    m2 = (labsel_ref[...][:, None] == cid).astype(jnp.float32)
    feats = jnp.minimum(
        jnp.dot(m1, m2, preferred_element_type=jnp.float32), 1.0)
    o_ref[:, :D] = x_ref[...]
    o_ref[:, D:] = feats


def _build_xc(x, idx_labeled, lab_sel):
    return pl.pallas_call(
        _xc_body,
        out_shape=jax.ShapeDtypeStruct((N, D + NC), jnp.float32),
        grid=(GRID,),
        in_specs=[
            pl.BlockSpec((ROWS, D), lambda i: (i, 0)),
            pl.BlockSpec((1000,), lambda i: (0,)),
            pl.BlockSpec((1000,), lambda i: (0,)),
        ],
        out_specs=pl.BlockSpec((ROWS, D + NC), lambda i: (i, 0)),
        name="tc_xc",
    )(x, idx_labeled, lab_sel)


def _mm_body(x_ref, w_ref, o_ref):
    o_ref[...] = jnp.dot(x_ref[...], w_ref[...],
                         preferred_element_type=jnp.float32)


def _mm(x, w):
    K = x.shape[1]
    M = w.shape[1]
    return pl.pallas_call(
        _mm_body,
        out_shape=jax.ShapeDtypeStruct((N, M), jnp.float32),
        grid=(GRID,),
        in_specs=[
            pl.BlockSpec((ROWS, K), lambda i: (i, 0)),
            pl.BlockSpec((K, M), lambda i: (0, 0)),
        ],
        out_specs=pl.BlockSpec((ROWS, M), lambda i: (i, 0)),
        name="tc_mm",
    )(x, w)


def _act_body(s_ref, b_ref, o_ref):
    o_ref[...] = jnp.tanh(s_ref[0] + s_ref[1] + b_ref[...][None, :])


def _act(parts, b):
    return pl.pallas_call(
        _act_body,
        out_shape=jax.ShapeDtypeStruct((N, NH), jnp.float32),
        grid=(GRID,),
        in_specs=[
            pl.BlockSpec((NSC, ROWS, NH), lambda i: (0, i, 0)),
            pl.BlockSpec((NH,), lambda i: (0,)),
        ],
        out_specs=pl.BlockSpec((ROWS, NH), lambda i: (i, 0)),
        name="tc_act",
    )(parts, b)


def _lsm_body(s_ref, b_ref, o_ref):
    t = s_ref[0] + s_ref[1] + b_ref[...][None, :]
    mask = lax.broadcasted_iota(jnp.int32, (ROWS, NH), 1) < NC
    t = jnp.where(mask, t, -jnp.inf)
    m = jnp.max(t, axis=1, keepdims=True)
    e = jnp.where(mask, jnp.exp(t - m), 0.0)
    lse = jnp.log(jnp.sum(e, axis=1, keepdims=True))
    o_ref[...] = (t - m - lse)[:, :NC]


def _lsm(parts, b_pad):
    return pl.pallas_call(
        _lsm_body,
        out_shape=jax.ShapeDtypeStruct((N, NC), jnp.float32),
        grid=(GRID,),
        in_specs=[
            pl.BlockSpec((NSC, ROWS, NH), lambda i: (0, i, 0)),
            pl.BlockSpec((NH,), lambda i: (0,)),
        ],
        out_specs=pl.BlockSpec((ROWS, NC), lambda i: (i, 0)),
        name="tc_lsm",
    )(parts, b_pad)


def kernel(x, W0, b0, W1, b1, W2, b2, W3, b3, W_out, b_out, edge_index,
           labels, idx_labeled):
    Ws = [W0, W1, W2, W3]
    bs = [b0, b1, b2, b3]
    srcs = jnp.pad(edge_index[0].reshape(NW, EPW), ((0, 0), (0, EPP - EPW)),
                   constant_values=0).reshape(NW * EPP)
    pad_dst = jnp.broadcast_to(jnp.arange(EPP - EPW, dtype=jnp.int32) + N,
                               (NW, EPP - EPW))
    dsts = jnp.concatenate([edge_index[1].reshape(NW, EPW), pad_dst],
                           axis=1).reshape(NW * EPP)
    lab_sel = jnp.take(labels, idx_labeled)
    zeros128 = jnp.zeros((NA, NH), jnp.float32)

    xc = _build_xc(x, idx_labeled, lab_sel)
    blocks = []
    for k in range(NL):
        inp = jnp.concatenate([xc] + blocks, axis=1) if blocks else xc
        xw = _mm(inp, Ws[k])
        parts = _spmm(xw, srcs, dsts, zeros128, W=NH)
        blocks.append(_act(parts, bs[k]))
    inp = jnp.concatenate([xc] + blocks, axis=1)
    w_pad = jnp.pad(W_out, ((0, 0), (0, NH - NC)))
    z = _mm(inp, w_pad)
    parts = _spmm(z, srcs, dsts, zeros128, W=NH)
    b_pad = jnp.pad(b_out, (0, NH - NC))
    return _lsm(parts, b_pad)


# speedup vs baseline: 1.8919x; 1.8919x over previous
import functools

import jax
import jax.numpy as jnp
from jax import lax
from jax.experimental import pallas as pl
from jax.experimental.pallas import tpu as pltpu
from jax.experimental.pallas import tpu_sc as plsc

N = 10000
E = 320000
D = 128
NC = 40
NH = 128
NL = 4

ROWS = 400
GRID = N // ROWS

NSC = 2
NSUB = 16
NW = NSC * NSUB
EPW = E // NW
B = 128
EPP = 10240
NBT = EPP // B
NQ = NBT // 4
NA = 10240
RPS = NA // NSUB


NB78 = EPW // B
TAIL = EPW - NB78 * B


def _spmm_body(W, xw, srcs, dsts, zeros, out, idx_s, idx_d, rows, idx_st,
               idx_dt, rows_t, acc, sem):
    c = lax.axis_index("c")
    s = lax.axis_index("s")
    row0 = s * RPS
    pltpu.sync_copy(zeros.at[pl.ds(row0, RPS)], acc.at[pl.ds(row0, RPS)])
    plsc.subcore_barrier()
    base = (c * NSUB + s) * EPW

    def step(j, carry):
        off = base + j * B
        pltpu.sync_copy(srcs.at[pl.ds(off, B)], idx_s)
        pltpu.sync_copy(dsts.at[pl.ds(off, B)], idx_d)
        pltpu.async_copy(xw.at[idx_s], rows, sem).wait()
        pltpu.sync_copy(rows, acc.at[idx_d], add=True)
        return carry

    lax.fori_loop(0, NB78, step, 0)
    off = base + NB78 * B
    pltpu.sync_copy(srcs.at[pl.ds(off, TAIL)], idx_st)
    pltpu.sync_copy(dsts.at[pl.ds(off, TAIL)], idx_dt)
    pltpu.async_copy(xw.at[idx_st], rows_t, sem).wait()
    pltpu.sync_copy(rows_t, acc.at[idx_dt], add=True)
    plsc.subcore_barrier()
    pltpu.sync_copy(acc.at[pl.ds(row0, RPS)], out.at[c, pl.ds(row0, RPS)])


@functools.partial(jax.jit, static_argnames=("W",))
def _spmm(xw, srcs, dsts, zeros, W):
    mesh = plsc.VectorSubcoreMesh(core_axis_name="c", subcore_axis_name="s")
    body = functools.partial(_spmm_body, W)
    return pl.kernel(
        body,
        out_type=jax.ShapeDtypeStruct((NSC, NA, W), jnp.float32),
        mesh=mesh,
        scratch_types=[
            pltpu.VMEM((B,), jnp.int32),
            pltpu.VMEM((B,), jnp.int32),
            pltpu.VMEM((B, W), jnp.float32),
            pltpu.VMEM((TAIL,), jnp.int32),
            pltpu.VMEM((TAIL,), jnp.int32),
            pltpu.VMEM((TAIL, W), jnp.float32),
            pltpu.VMEM_SHARED((NA, W), jnp.float32),
            pltpu.SemaphoreType.DMA,
        ],
        name=f"sc_spmm_{W}",
    )(xw, srcs, dsts, zeros)


def _xc_body(x_ref, idxl_ref, labsel_ref, o_ref):
    i = pl.program_id(0)
    rid = lax.broadcasted_iota(jnp.int32, (ROWS, 1000), 0) + i * ROWS
    m1 = (rid == idxl_ref[...][None, :]).astype(jnp.float32)
    cid = lax.broadcasted_iota(jnp.int32, (1000, NC), 1)
    m2 = (labsel_ref[...][:, None] == cid).astype(jnp.float32)
    feats = jnp.minimum(
        jnp.dot(m1, m2, preferred_element_type=jnp.float32), 1.0)
    o_ref[:, :D] = x_ref[...]
    o_ref[:, D:] = feats


def _build_xc(x, idx_labeled, lab_sel):
    return pl.pallas_call(
        _xc_body,
        out_shape=jax.ShapeDtypeStruct((N, D + NC), jnp.float32),
        grid=(GRID,),
        in_specs=[
            pl.BlockSpec((ROWS, D), lambda i: (i, 0)),
            pl.BlockSpec((1000,), lambda i: (0,)),
            pl.BlockSpec((1000,), lambda i: (0,)),
        ],
        out_specs=pl.BlockSpec((ROWS, D + NC), lambda i: (i, 0)),
        name="tc_xc",
    )(x, idx_labeled, lab_sel)


def _mm_body(x_ref, w_ref, o_ref):
    o_ref[...] = jnp.dot(x_ref[...], w_ref[...],
                         preferred_element_type=jnp.float32)


def _mm(x, w):
    K = x.shape[1]
    M = w.shape[1]
    return pl.pallas_call(
        _mm_body,
        out_shape=jax.ShapeDtypeStruct((N, M), jnp.float32),
        grid=(GRID,),
        in_specs=[
            pl.BlockSpec((ROWS, K), lambda i: (i, 0)),
            pl.BlockSpec((K, M), lambda i: (0, 0)),
        ],
        out_specs=pl.BlockSpec((ROWS, M), lambda i: (i, 0)),
        name="tc_mm",
    )(x, w)


def _act_body(s_ref, b_ref, o_ref):
    o_ref[...] = jnp.tanh(s_ref[0] + s_ref[1] + b_ref[...][None, :])


def _act(parts, b):
    return pl.pallas_call(
        _act_body,
        out_shape=jax.ShapeDtypeStruct((N, NH), jnp.float32),
        grid=(GRID,),
        in_specs=[
            pl.BlockSpec((NSC, ROWS, NH), lambda i: (0, i, 0)),
            pl.BlockSpec((NH,), lambda i: (0,)),
        ],
        out_specs=pl.BlockSpec((ROWS, NH), lambda i: (i, 0)),
        name="tc_act",
    )(parts, b)


def _lsm_body(s_ref, b_ref, o_ref):
    t = s_ref[0] + s_ref[1] + b_ref[...][None, :]
    mask = lax.broadcasted_iota(jnp.int32, (ROWS, NH), 1) < NC
    t = jnp.where(mask, t, -jnp.inf)
    m = jnp.max(t, axis=1, keepdims=True)
    e = jnp.where(mask, jnp.exp(t - m), 0.0)
    lse = jnp.log(jnp.sum(e, axis=1, keepdims=True))
    o_ref[...] = (t - m - lse)[:, :NC]


def _lsm(parts, b_pad):
    return pl.pallas_call(
        _lsm_body,
        out_shape=jax.ShapeDtypeStruct((N, NC), jnp.float32),
        grid=(GRID,),
        in_specs=[
            pl.BlockSpec((NSC, ROWS, NH), lambda i: (0, i, 0)),
            pl.BlockSpec((NH,), lambda i: (0,)),
        ],
        out_specs=pl.BlockSpec((ROWS, NC), lambda i: (i, 0)),
        name="tc_lsm",
    )(parts, b_pad)


def kernel(x, W0, b0, W1, b1, W2, b2, W3, b3, W_out, b_out, edge_index,
           labels, idx_labeled):
    Ws = [W0, W1, W2, W3]
    bs = [b0, b1, b2, b3]
    srcs = edge_index[0]
    dsts = edge_index[1]
    lab_sel = jnp.take(labels, idx_labeled)
    zeros128 = jnp.zeros((NA, NH), jnp.float32)

    xc = _build_xc(x, idx_labeled, lab_sel)
    blocks = []
    for k in range(NL):
        inp = jnp.concatenate([xc] + blocks, axis=1) if blocks else xc
        xw = _mm(inp, Ws[k])
        parts = _spmm(xw, srcs, dsts, zeros128, W=NH)
        blocks.append(_act(parts, bs[k]))
    inp = jnp.concatenate([xc] + blocks, axis=1)
    w_pad = jnp.pad(W_out, ((0, 0), (0, NH - NC)))
    z = _mm(inp, w_pad)
    parts = _spmm(z, srcs, dsts, zeros128, W=NH)
    b_pad = jnp.pad(b_out, (0, NH - NC))
    return _lsm(parts, b_pad)


# speedup vs baseline: 2.0742x; 1.0963x over previous
import functools

import jax
import jax.numpy as jnp
from jax import lax
from jax.experimental import pallas as pl
from jax.experimental.pallas import tpu as pltpu
from jax.experimental.pallas import tpu_sc as plsc

N = 10000
E = 320000
D = 128
NC = 40
NH = 128
NL = 4

ROWS = 400
GRID = N // ROWS

NSC = 2
NSUB = 16
NW = NSC * NSUB
EPW = E // NW
B = 128
EPP = 10240
NBT = EPP // B
NQ = NBT // 4
NA = 10240
RPS = NA // NSUB


NB78 = EPW // B
TAIL = EPW - NB78 * B


def _spmm_body(W, xw, srcs, dsts, zeros, out, idx_s, idx_d, rows, idx_st,
               idx_dt, rows_t, acc, sem):
    c = lax.axis_index("c")
    s = lax.axis_index("s")
    row0 = s * RPS
    pltpu.sync_copy(zeros.at[pl.ds(row0, RPS)], acc.at[pl.ds(row0, RPS)])
    plsc.subcore_barrier()
    base = (c * NSUB + s) * EPW

    def step(j, carry):
        off = base + j * B
        pltpu.sync_copy(srcs.at[pl.ds(off, B)], idx_s)
        pltpu.sync_copy(dsts.at[pl.ds(off, B)], idx_d)
        pltpu.async_copy(xw.at[idx_s], rows, sem).wait()
        pltpu.sync_copy(rows, acc.at[idx_d], add=True)
        return carry

    lax.fori_loop(0, NB78, step, 0)
    off = base + NB78 * B
    pltpu.sync_copy(srcs.at[pl.ds(off, TAIL)], idx_st)
    pltpu.sync_copy(dsts.at[pl.ds(off, TAIL)], idx_dt)
    pltpu.async_copy(xw.at[idx_st], rows_t, sem).wait()
    pltpu.sync_copy(rows_t, acc.at[idx_dt], add=True)
    plsc.subcore_barrier()
    pltpu.sync_copy(acc.at[pl.ds(row0, RPS)], out.at[c, pl.ds(row0, RPS)])


@functools.partial(jax.jit, static_argnames=("W",))
def _spmm(xw, srcs, dsts, zeros, W):
    mesh = plsc.VectorSubcoreMesh(core_axis_name="c", subcore_axis_name="s")
    body = functools.partial(_spmm_body, W)
    return pl.kernel(
        body,
        out_type=jax.ShapeDtypeStruct((NSC, NA, W), jnp.float32),
        mesh=mesh,
        scratch_types=[
            pltpu.VMEM((B,), jnp.int32),
            pltpu.VMEM((B,), jnp.int32),
            pltpu.VMEM((B, W), jnp.float32),
            pltpu.VMEM((TAIL,), jnp.int32),
            pltpu.VMEM((TAIL,), jnp.int32),
            pltpu.VMEM((TAIL, W), jnp.float32),
            pltpu.VMEM_SHARED((NA, W), jnp.float32),
            pltpu.SemaphoreType.DMA,
        ],
        name=f"sc_spmm_{W}",
    )(xw, srcs, dsts, zeros)


def _xc_body(x_ref, idxl_ref, labsel_ref, o_ref):
    i = pl.program_id(0)
    rid = lax.broadcasted_iota(jnp.int32, (ROWS, 1000), 0) + i * ROWS
    m1 = (rid == idxl_ref[...][None, :]).astype(jnp.float32)
    cid = lax.broadcasted_iota(jnp.int32, (1000, NC), 1)
    m2 = (labsel_ref[...][:, None] == cid).astype(jnp.float32)
    feats = jnp.minimum(
        jnp.dot(m1, m2, preferred_element_type=jnp.float32), 1.0)
    o_ref[:, :D] = x_ref[...]
    o_ref[:, D:] = feats


def _build_xc(x, idx_labeled, lab_sel):
    return pl.pallas_call(
        _xc_body,
        out_shape=jax.ShapeDtypeStruct((N, D + NC), jnp.float32),
        grid=(GRID,),
        in_specs=[
            pl.BlockSpec((ROWS, D), lambda i: (i, 0)),
            pl.BlockSpec((1000,), lambda i: (0,)),
            pl.BlockSpec((1000,), lambda i: (0,)),
        ],
        out_specs=pl.BlockSpec((ROWS, D + NC), lambda i: (i, 0)),
        name="tc_xc",
    )(x, idx_labeled, lab_sel)


def _mm_body(x_ref, w_ref, o_ref):
    o_ref[...] = jnp.dot(x_ref[...], w_ref[...],
                         preferred_element_type=jnp.float32)


def _mm(x, w):
    K = x.shape[1]
    M = w.shape[1]
    return pl.pallas_call(
        _mm_body,
        out_shape=jax.ShapeDtypeStruct((N, M), jnp.float32),
        grid=(GRID,),
        in_specs=[
            pl.BlockSpec((ROWS, K), lambda i: (i, 0)),
            pl.BlockSpec((K, M), lambda i: (0, 0)),
        ],
        out_specs=pl.BlockSpec((ROWS, M), lambda i: (i, 0)),
        name="tc_mm",
    )(x, w)


def _layer(k, last, xc, blocks, parts, bias, w):
    nprev = len(blocks)

    def body(*refs):
        xc_ref = refs[0]
        blks = refs[1:1 + nprev]
        p_ref = refs[1 + nprev]
        b_ref = refs[2 + nprev]
        w_ref = refs[3 + nprev]
        outs = refs[4 + nprev:]
        blk = jnp.tanh(p_ref[0] + p_ref[1] + b_ref[...][None, :])
        acc = jnp.dot(xc_ref[...], w_ref[pl.ds(0, D + NC), :],
                      preferred_element_type=jnp.float32)
        off = D + NC
        for br in blks:
            acc += jnp.dot(br[...], w_ref[pl.ds(off, NH), :],
                           preferred_element_type=jnp.float32)
            off += NH
        acc += jnp.dot(blk, w_ref[pl.ds(off, NH), :],
                       preferred_element_type=jnp.float32)
        outs[0][...] = acc
        if not last:
            outs[1][...] = blk

    K, M = w.shape
    out_shape = [jax.ShapeDtypeStruct((N, M), jnp.float32)]
    out_specs = [pl.BlockSpec((ROWS, M), lambda i: (i, 0))]
    if not last:
        out_shape.append(jax.ShapeDtypeStruct((N, NH), jnp.float32))
        out_specs.append(pl.BlockSpec((ROWS, NH), lambda i: (i, 0)))
    return pl.pallas_call(
        body,
        out_shape=out_shape,
        grid=(GRID,),
        in_specs=[pl.BlockSpec((ROWS, D + NC), lambda i: (i, 0))]
        + [pl.BlockSpec((ROWS, NH), lambda i: (i, 0))] * nprev
        + [
            pl.BlockSpec((NSC, ROWS, NH), lambda i: (0, i, 0)),
            pl.BlockSpec((NH,), lambda i: (0,)),
            pl.BlockSpec((K, M), lambda i: (0, 0)),
        ],
        out_specs=out_specs,
        name=f"tc_layer{k}",
    )(xc, *blocks, parts, bias, w)


def _lsm_body(s_ref, b_ref, o_ref):
    t = s_ref[0] + s_ref[1] + b_ref[...][None, :]
    mask = lax.broadcasted_iota(jnp.int32, (ROWS, NH), 1) < NC
    t = jnp.where(mask, t, -jnp.inf)
    m = jnp.max(t, axis=1, keepdims=True)
    e = jnp.where(mask, jnp.exp(t - m), 0.0)
    lse = jnp.log(jnp.sum(e, axis=1, keepdims=True))
    o_ref[...] = (t - m - lse)[:, :NC]


def _lsm(parts, b_pad):
    return pl.pallas_call(
        _lsm_body,
        out_shape=jax.ShapeDtypeStruct((N, NC), jnp.float32),
        grid=(GRID,),
        in_specs=[
            pl.BlockSpec((NSC, ROWS, NH), lambda i: (0, i, 0)),
            pl.BlockSpec((NH,), lambda i: (0,)),
        ],
        out_specs=pl.BlockSpec((ROWS, NC), lambda i: (i, 0)),
        name="tc_lsm",
    )(parts, b_pad)


def kernel(x, W0, b0, W1, b1, W2, b2, W3, b3, W_out, b_out, edge_index,
           labels, idx_labeled):
    Ws = [W0, W1, W2, W3]
    bs = [b0, b1, b2, b3]
    srcs = edge_index[0]
    dsts = edge_index[1]
    lab_sel = jnp.take(labels, idx_labeled)
    zeros128 = jnp.zeros((NA, NH), jnp.float32)

    xc = _build_xc(x, idx_labeled, lab_sel)
    xw = _mm(xc, W0)
    parts = _spmm(xw, srcs, dsts, zeros128, W=NH)
    blocks = []
    for k in range(1, NL):
        xw, blk = _layer(k, False, xc, blocks, parts, bs[k - 1], Ws[k])
        blocks.append(blk)
        parts = _spmm(xw, srcs, dsts, zeros128, W=NH)
    w_pad = jnp.pad(W_out, ((0, 0), (0, NH - NC)))
    z = _layer(NL, True, xc, blocks, parts, bs[NL - 1], w_pad)[0]
    parts = _spmm(z, srcs, dsts, zeros128, W=NH)
    b_pad = jnp.pad(b_out, (0, NH - NC))
    return _lsm(parts, b_pad)
